# baseline (speedup 1.0000x reference)
import functools
import math

import jax
import jax.numpy as jnp
from jax import lax
from jax.experimental import pallas as pl
from jax.experimental.pallas import tpu as pltpu
from jax.experimental.pallas import tpu_sc as plsc

_NUM_CORES = 2
_NUM_SUBCORES = 16
_NUM_WORKERS = _NUM_CORES * _NUM_SUBCORES
_LANES = 16

_CHUNK = 8
_NSLOT = 3
_NPOS = 2
_COLG = 4


def _build(B, S, V, D):
    s_per_w = S // _NUM_WORKERS
    n_chunks = s_per_w // _CHUNK
    scale = jnp.float32(math.sqrt(D))
    vecs_per_row = D // _LANES
    vecs_per_cg = vecs_per_row // _COLG

    mesh = plsc.VectorSubcoreMesh(core_axis_name="c", subcore_axis_name="s")

    row_bufs = [pltpu.VMEM((_CHUNK, D), jnp.float32)
                for _ in range(_NSLOT * B)]
    pos_bufs = [pltpu.VMEM((_CHUNK, D), jnp.float32) for _ in range(_NPOS)]
    sems = [pltpu.SemaphoreType.DMA for _ in range(2 * _NSLOT + _NPOS)]

    @functools.partial(
        pl.kernel,
        mesh=mesh,
        out_type=jax.ShapeDtypeStruct((B, S, D), jnp.float32),
        scratch_types=[pltpu.VMEM((B, s_per_w), jnp.int32)]
        + row_bufs + pos_bufs + sems,
    )
    def embed(tok_hbm, ttab_hbm, ptab_hbm, out_hbm, idx_v, *bufs):
        row_v = [bufs[s * B:(s + 1) * B] for s in range(_NSLOT)]
        pos_v = bufs[_NSLOT * B:_NSLOT * B + _NPOS]
        g_sem = bufs[_NSLOT * B + _NPOS:_NSLOT * B + _NPOS + _NSLOT]
        w_sem = bufs[_NSLOT * B + _NPOS + _NSLOT:_NSLOT * B + _NPOS + 2 * _NSLOT]
        p_sem = bufs[_NSLOT * B + _NPOS + 2 * _NSLOT:]

        wid = lax.axis_index("s") * _NUM_CORES + lax.axis_index("c")
        s_base = wid * s_per_w

        pltpu.sync_copy(tok_hbm.at[:, pl.ds(s_base, s_per_w)], idx_v)

        def pos_load(j):
            return pltpu.async_copy(
                ptab_hbm.at[pl.ds(s_base + j * _CHUNK, _CHUNK)],
                pos_v[j % _NPOS], p_sem[j % _NPOS])

        def gather(j, b):
            idx = idx_v.at[b, pl.ds(j * _CHUNK, _CHUNK)]
            return pltpu.async_copy(ttab_hbm.at[idx], row_v[j % _NSLOT][b],
                                    g_sem[j % _NSLOT])

        def write(j, b):
            return pltpu.async_copy(
                row_v[j % _NSLOT][b],
                out_hbm.at[b, pl.ds(s_base + j * _CHUNK, _CHUNK)],
                w_sem[j % _NSLOT])

        pos_h = {0: pos_load(0)}
        g_h = {0: [gather(0, b) for b in range(B)]}
        w_h = {}

        for j in range(n_chunks):
            nj = j + 1
            if nj < n_chunks:
                if nj >= _NSLOT:
                    for h in w_h[nj - _NSLOT]:
                        h.wait()
                g_h[nj] = [gather(nj, b) for b in range(B)]
                pos_h[nj] = pos_load(nj)
            for h in g_h[j]:
                h.wait()
            pos_h[j].wait()

            rvs, pv = row_v[j % _NSLOT], pos_v[j % _NPOS]

            def row_body(r, _, rvs=rvs, pv=pv):
                def cg_body(cg, _):
                    base = cg * (vecs_per_cg * _LANES)
                    for c in range(vecs_per_cg):
                        sl = pl.ds(base + c * _LANES, _LANES)
                        p = pv[r, sl]
                        for rv in rvs:
                            rv[r, sl] = (rv[r, sl] + p) * scale
                    return _

                lax.fori_loop(0, _COLG, cg_body, 0)
                return _

            lax.fori_loop(0, _CHUNK, row_body, 0)
            w_h[j] = [write(j, b) for b in range(B)]

        for j in range(max(0, n_chunks - _NSLOT), n_chunks):
            for h in w_h[j]:
                h.wait()

    return embed


def kernel(tokens, token_table, pos_table):
    B, S = tokens.shape
    V, D = token_table.shape
    embed = _build(B, S, V, D)
    return embed(tokens.astype(jnp.int32), token_table, pos_table)


# speedup vs baseline: 2.3169x; 2.3169x over previous
import functools
import math

import jax
import jax.numpy as jnp
from jax import lax
from jax.experimental import pallas as pl
from jax.experimental.pallas import tpu as pltpu
from jax.experimental.pallas import tpu_sc as plsc

_NUM_CORES = 2
_NUM_SUBCORES = 16
_NUM_WORKERS = _NUM_CORES * _NUM_SUBCORES
_LANES = 16

_CHUNK = 16
_NG = 3
_NW = 2
_NPOS = 2
_AHEAD = 2


def _build(B, S, V, D):
    s_per_w = S // _NUM_WORKERS
    n_chunks = s_per_w // _CHUNK
    n_units = n_chunks * B
    scale = jnp.float32(math.sqrt(D))
    vecs_per_row = D // _LANES

    mesh = plsc.VectorSubcoreMesh(core_axis_name="c", subcore_axis_name="s")

    bufs_t = (
        [pltpu.VMEM((_CHUNK, D), jnp.float32) for _ in range(_NG)]
        + [pltpu.VMEM((_CHUNK, D), jnp.float32) for _ in range(_NW)]
        + [pltpu.VMEM((_CHUNK, D), jnp.float32) for _ in range(_NPOS)]
        + [pltpu.SemaphoreType.DMA for _ in range(_NG + _NW + _NPOS)]
    )

    @functools.partial(
        pl.kernel,
        mesh=mesh,
        out_type=jax.ShapeDtypeStruct((B, S, D), jnp.float32),
        scratch_types=[pltpu.VMEM((B, s_per_w), jnp.int32)] + bufs_t,
    )
    def embed(tok_hbm, ttab_hbm, ptab_hbm, out_hbm, idx_v, *bufs):
        row_v = bufs[:_NG]
        out_v = bufs[_NG:_NG + _NW]
        pos_v = bufs[_NG + _NW:_NG + _NW + _NPOS]
        n_b = _NG + _NW + _NPOS
        g_sem = bufs[n_b:n_b + _NG]
        w_sem = bufs[n_b + _NG:n_b + _NG + _NW]
        p_sem = bufs[n_b + _NG + _NW:]

        wid = lax.axis_index("s") * _NUM_CORES + lax.axis_index("c")
        s_base = wid * s_per_w

        pltpu.sync_copy(tok_hbm.at[:, pl.ds(s_base, s_per_w)], idx_v)

        def pos_load(j):
            return pltpu.async_copy(
                ptab_hbm.at[pl.ds(s_base + j * _CHUNK, _CHUNK)],
                pos_v[j % _NPOS], p_sem[j % _NPOS])

        def gather(u):
            j, b = divmod(u, B)
            idx = idx_v.at[b, pl.ds(j * _CHUNK, _CHUNK)]
            return pltpu.async_copy(ttab_hbm.at[idx], row_v[u % _NG],
                                    g_sem[u % _NG])

        def write(u):
            j, b = divmod(u, B)
            return pltpu.async_copy(
                out_v[u % _NW],
                out_hbm.at[b, pl.ds(s_base + j * _CHUNK, _CHUNK)],
                w_sem[u % _NW])

        pos_h = {j: pos_load(j) for j in range(min(_NPOS, n_chunks))}
        g_h = {u: gather(u) for u in range(min(_AHEAD + 1, n_units))}
        w_h = {}

        for u in range(n_units):
            j, b = divmod(u, B)
            nu = u + _AHEAD
            if nu < n_units and nu > _AHEAD:
                g_h[nu] = gather(nu)
            g_h[u].wait()
            if b == 0:
                pos_h[j].wait()
            if u >= _NW:
                w_h[u - _NW].wait()

            rv, ov, pv = row_v[u % _NG], out_v[u % _NW], pos_v[j % _NPOS]

            def row_body(r, _, rv=rv, ov=ov, pv=pv):
                for c in range(vecs_per_row):
                    sl = pl.ds(c * _LANES, _LANES)
                    ov[r, sl] = (rv[r, sl] + pv[r, sl]) * scale
                return _

            lax.fori_loop(0, _CHUNK, row_body, 0)
            w_h[u] = write(u)
            if b == B - 1 and j + _NPOS < n_chunks:
                pos_h[j + _NPOS] = pos_load(j + _NPOS)

        for u in range(max(0, n_units - _NW), n_units):
            w_h[u].wait()

    return embed


def kernel(tokens, token_table, pos_table):
    B, S = tokens.shape
    V, D = token_table.shape
    embed = _build(B, S, V, D)
    return embed(tokens.astype(jnp.int32), token_table, pos_table)
